# baseline (speedup 1.0000x reference)
import jax
import jax.numpy as jnp
from jax.experimental import pallas as pl
from jax.experimental.pallas import tpu as pltpu

_BINS = 16


def _dfl_body(x_ref, d_ref, m_ref, tot_ref, npos_ref):
    first = jnp.logical_and(pl.program_id(0) == 0, pl.program_id(1) == 0)
    pm = m_ref[0].astype(jnp.float32)
    partial = jnp.zeros((), jnp.float32)
    for s in range(4):
        d = jnp.clip(d_ref[0, s], 0.0, float(_BINS - 1))
        base = s * _BINS
        mx = x_ref[0, base]
        for k in range(1, _BINS):
            mx = jnp.maximum(mx, x_ref[0, base + k])
        xp = x_ref[0, base]
        ssum = jnp.exp(xp - mx)
        acc = xp
        for k in range(1, _BINS):
            xk = x_ref[0, base + k]
            ssum += jnp.exp(xk - mx)
            acc += jnp.clip(d - float(k - 1), 0.0, 1.0) * (xk - xp)
            xp = xk
        lse = jnp.log(ssum) + mx
        partial += jnp.sum((lse - acc) * pm)

    @pl.when(first)
    def _init():
        tot_ref[0, 0] = 0.0
        npos_ref[0, 0] = 0.0

    tot_ref[0, 0] += partial
    npos_ref[0, 0] += jnp.sum(pm)


@jax.jit
def kernel(reg_logits, dist_targets, pos_mask):
    B, C, H, W = reg_logits.shape
    dist_t = jnp.transpose(dist_targets, (0, 2, 1)).reshape(B, 4, H, W)
    pm = pos_mask.reshape(B, H, W)

    tot, npos = pl.pallas_call(
        _dfl_body,
        grid=(B, 2),
        in_specs=[
            pl.BlockSpec((1, C, H // 2, W), lambda b, h: (b, 0, h, 0)),
            pl.BlockSpec((1, 4, H // 2, W), lambda b, h: (b, 0, h, 0)),
            pl.BlockSpec((1, H // 2, W), lambda b, h: (b, h, 0)),
        ],
        out_specs=[
            pl.BlockSpec(memory_space=pltpu.SMEM),
            pl.BlockSpec(memory_space=pltpu.SMEM),
        ],
        out_shape=[
            jax.ShapeDtypeStruct((1, 1), jnp.float32),
            jax.ShapeDtypeStruct((1, 1), jnp.float32),
        ],
    )(reg_logits, dist_t, pm)

    total = tot[0, 0]
    n_pos = npos[0, 0]
    return jnp.where(n_pos > 0, total / jnp.maximum(n_pos * 4.0, 1.0), 0.0)


# speedup vs baseline: 1.1684x; 1.1684x over previous
import jax
import jax.numpy as jnp
from jax.experimental import pallas as pl
from jax.experimental.pallas import tpu as pltpu

_BINS = 16


def _dfl_body(x_ref, d_ref, m_ref, tot_ref, npos_ref):
    b = pl.program_id(0)
    pm = m_ref[0].astype(jnp.float32)
    partial = jnp.zeros((), jnp.float32)
    for s in range(4):
        d = jnp.clip(d_ref[0, s], 0.0, float(_BINS - 1))
        base = s * _BINS
        mx = x_ref[0, base]
        for k in range(1, _BINS):
            mx = jnp.maximum(mx, x_ref[0, base + k])
        xp = x_ref[0, base]
        ssum = jnp.exp(xp - mx)
        acc = xp
        for k in range(1, _BINS):
            xk = x_ref[0, base + k]
            ssum += jnp.exp(xk - mx)
            acc += jnp.clip(d - float(k - 1), 0.0, 1.0) * (xk - xp)
            xp = xk
        lse = jnp.log(ssum) + mx
        partial += jnp.sum((lse - acc) * pm)

    @pl.when(b == 0)
    def _init():
        tot_ref[0, 0] = 0.0
        npos_ref[0, 0] = 0.0

    tot_ref[0, 0] += partial
    npos_ref[0, 0] += jnp.sum(pm)


@jax.jit
def kernel(reg_logits, dist_targets, pos_mask):
    B, C, H, W = reg_logits.shape
    dist_t = jnp.transpose(dist_targets, (0, 2, 1)).reshape(B, 4, H, W)
    pm = pos_mask.reshape(B, H, W)

    tot, npos = pl.pallas_call(
        _dfl_body,
        grid=(B,),
        in_specs=[
            pl.BlockSpec((1, C, H, W), lambda b: (b, 0, 0, 0)),
            pl.BlockSpec((1, 4, H, W), lambda b: (b, 0, 0, 0)),
            pl.BlockSpec((1, H, W), lambda b: (b, 0, 0)),
        ],
        out_specs=[
            pl.BlockSpec(memory_space=pltpu.SMEM),
            pl.BlockSpec(memory_space=pltpu.SMEM),
        ],
        out_shape=[
            jax.ShapeDtypeStruct((1, 1), jnp.float32),
            jax.ShapeDtypeStruct((1, 1), jnp.float32),
        ],
    )(reg_logits, dist_t, pm)

    total = tot[0, 0]
    n_pos = npos[0, 0]
    return jnp.where(n_pos > 0, total / jnp.maximum(n_pos * 4.0, 1.0), 0.0)
